# baseline (speedup 1.0000x reference)
import functools

import jax
import jax.numpy as jnp
from jax import lax
from jax.experimental import pallas as pl
from jax.experimental.pallas import tpu as pltpu
from jax.experimental.pallas import tpu_sc as plsc

N = 10000
NPAD = 10240
E = 320000
CHUNK = 128
NCHUNKS = 2560
CPW = NCHUNKS // 32
ROWS_PER_SUB = NPAD // 16

_MESH = plsc.VectorSubcoreMesh(core_axis_name="c", subcore_axis_name="s")
_SC_PARAMS = pltpu.CompilerParams(use_tc_tiling_on_sc=False)


@functools.partial(
    pl.kernel,
    out_type=jax.ShapeDtypeStruct((NPAD, 32), jnp.float32),
    mesh=_MESH,
    compiler_params=_SC_PARAMS,
    scratch_types=[
        pltpu.VMEM_SHARED((NPAD, 16), jnp.float32),
        pltpu.VMEM((CPW, CHUNK), jnp.int32),
        pltpu.VMEM((CHUNK, 16), jnp.float32),
    ],
)
def _sc_deg(dst_hbm, ones_hbm, zeros_hbm, out, acc, idx_v, ones_v):
    cid = lax.axis_index("c")
    sid = lax.axis_index("s")
    sl = pl.ds(sid * ROWS_PER_SUB, ROWS_PER_SUB)
    pltpu.sync_copy(zeros_hbm.at[sl], acc.at[sl])
    pltpu.sync_copy(ones_hbm, ones_v)
    w = cid * 16 + sid
    pltpu.sync_copy(dst_hbm.at[pl.ds(w * CPW, CPW)], idx_v)
    plsc.subcore_barrier()

    @pl.loop(0, CPW)
    def _(i):
        pltpu.sync_copy(ones_v, acc.at[idx_v.at[i]], add=True)

    plsc.subcore_barrier()

    @pl.when(cid == 0)
    def _():
        pltpu.sync_copy(acc.at[sl], out.at[sl, pl.ds(0, 16)])

    @pl.when(cid == 1)
    def _():
        pltpu.sync_copy(acc.at[sl], out.at[sl, pl.ds(16, 16)])


BLK = 40
CPW0 = 80
CPW1 = 80


def _make_sc_agg(F, packed):
    out_type = (jax.ShapeDtypeStruct((NPAD, 2 * F), jnp.float32) if packed
                else (jax.ShapeDtypeStruct((NPAD, F), jnp.float32),
                      jax.ShapeDtypeStruct((NPAD, F), jnp.float32)))

    @functools.partial(
        pl.kernel,
        out_type=out_type,
        mesh=_MESH,
        compiler_params=_SC_PARAMS,
        scratch_types=[
            pltpu.VMEM_SHARED((NPAD, F), jnp.float32),
            pltpu.VMEM((BLK, CHUNK), jnp.int32),
            pltpu.VMEM((BLK, CHUNK), jnp.int32),
            pltpu.VMEM((CHUNK, F), jnp.float32),
            pltpu.VMEM((CHUNK, F), jnp.float32),
            pltpu.SemaphoreType.DMA,
            pltpu.SemaphoreType.DMA,
        ],
    )
    def k(tab_hbm, src_hbm, dst_hbm, zeros_hbm, *out_and_scratch):
        if packed:
            (out, acc, si_v, di_v, rows_a, rows_b,
             sem_a, sem_b) = out_and_scratch
        else:
            (out0, out1, acc, si_v, di_v, rows_a, rows_b,
             sem_a, sem_b) = out_and_scratch
        cid = lax.axis_index("c")
        sid = lax.axis_index("s")
        sl = pl.ds(sid * ROWS_PER_SUB, ROWS_PER_SUB)
        pltpu.sync_copy(zeros_hbm.at[sl], acc.at[sl])
        plsc.subcore_barrier()

        def run(worker_base, nblk):
            @pl.loop(0, nblk)
            def _(h):
                base = worker_base + h * BLK
                pltpu.sync_copy(src_hbm.at[pl.ds(base, BLK)], si_v)
                pltpu.sync_copy(dst_hbm.at[pl.ds(base, BLK)], di_v)
                pltpu.async_copy(tab_hbm.at[si_v.at[0]], rows_a, sem_a)

                @pl.loop(0, BLK, step=2)
                def _(i):
                    pltpu.async_copy(tab_hbm.at[si_v.at[i + 1]], rows_b, sem_b)
                    pltpu.make_async_copy(tab_hbm.at[si_v.at[i]], rows_a,
                                          sem_a).wait()
                    pltpu.sync_copy(rows_a, acc.at[di_v.at[i]], add=True)

                    @pl.when(i + 2 < BLK)
                    def _():
                        pltpu.async_copy(tab_hbm.at[si_v.at[i + 2]], rows_a,
                                         sem_a)

                    pltpu.make_async_copy(tab_hbm.at[si_v.at[i + 1]], rows_b,
                                          sem_b).wait()
                    pltpu.sync_copy(rows_b, acc.at[di_v.at[i + 1]], add=True)

        @pl.when(cid == 0)
        def _():
            run(sid * CPW0, CPW0 // BLK)

        @pl.when(cid == 1)
        def _():
            run(16 * CPW0 + sid * CPW1, CPW1 // BLK)

        plsc.subcore_barrier()

        if packed:
            @pl.when(cid == 0)
            def _():
                pltpu.sync_copy(acc.at[sl], out.at[sl, pl.ds(0, F)])

            @pl.when(cid == 1)
            def _():
                pltpu.sync_copy(acc.at[sl], out.at[sl, pl.ds(F, F)])
        else:
            @pl.when(cid == 0)
            def _():
                pltpu.sync_copy(acc.at[sl], out0.at[sl])

            @pl.when(cid == 1)
            def _():
                pltpu.sync_copy(acc.at[sl], out1.at[sl])

    return k


_sc_agg64 = _make_sc_agg(64, packed=True)
_sc_agg128 = _make_sc_agg(128, packed=False)


def _row_mask():
    return lax.broadcasted_iota(jnp.int32, (NPAD, 1), 0) < N


def _bn(z, g, b, mask, eps=1e-5):
    zm = jnp.where(mask, z, 0.0)
    m = jnp.sum(zm, axis=0, keepdims=True) * (1.0 / N)
    d = jnp.where(mask, z - m, 0.0)
    v = jnp.sum(d * d, axis=0, keepdims=True) * (1.0 / N)
    return (z - m) * lax.rsqrt(v + eps) * g + b


def _dot(a, b):
    return jnp.dot(a, b, preferred_element_type=jnp.float32)


def _tc_h0_body(x_ref, w_ref, b_ref, o_ref):
    o_ref[...] = jnp.maximum(_dot(x_ref[...], w_ref[...]) + b_ref[...], 0.0)


def _tc_prep1_body(h0_ref, d_ref, dis_ref, t1_ref):
    deg = d_ref[:, 0:1] + d_ref[:, 16:17] + 1.0
    dis = jnp.where(_row_mask(), lax.rsqrt(deg), 0.0)
    dis_ref[...] = dis
    t1_ref[...] = dis * h0_ref[...]


def _tc_mid1_body(apk, t1, dis_r, w1, b1, g1, be1, w2, t2_ref):
    mask = _row_mask()
    dis = dis_r[...]
    u = dis * (apk[:, :64] + apk[:, 64:] + t1[...])
    z = _dot(u, w1[...]) + b1[...]
    h = jnp.maximum(_bn(z, g1[...], be1[...], mask), 0.0)
    t2_ref[...] = dis * _dot(h, w2[...])


def _tc_mid2_body(a0, a1, t2, dis_r, b2, g2, be2, w3, t3_ref):
    mask = _row_mask()
    dis = dis_r[...]
    z = dis * (a0[...] + a1[...] + t2[...]) + b2[...]
    h = jnp.maximum(_bn(z, g2[...], be2[...], mask), 0.0)
    t3_ref[...] = dis * _dot(h, w3[...])


def _tc_final_body(apk, t3, dis_r, b3, g3, be3, wo, bo, out_ref):
    mask = _row_mask()
    dis = dis_r[...]
    z = dis * (apk[:, :64] + apk[:, 64:] + t3[...]) + b3[...]
    h = jnp.maximum(_bn(z, g3[...], be3[...], mask), 0.0)
    logits = _dot(h, wo[...]) + bo[...]
    m = jnp.max(logits, axis=1, keepdims=True)
    lse = m + jnp.log(jnp.sum(jnp.exp(logits - m), axis=1, keepdims=True))
    out_ref[...] = logits - lse


def _tc_call(body, out_shapes, *args):
    return pl.pallas_call(
        body,
        out_shape=out_shapes,
    )(*args)


def kernel(x, edge_index, W_in, b_in, W1, b1, g1, be1, W2, b2, g2, be2,
           W3, b3, g3, be3, W_out, b_out):
    f32 = jnp.float32
    pad = NCHUNKS * CHUNK - E
    fill = N + (jnp.arange(pad, dtype=jnp.int32) % CHUNK)
    src_p = jnp.concatenate([edge_index[0], fill]).reshape(NCHUNKS, CHUNK)
    dst_p = jnp.concatenate([edge_index[1], fill]).reshape(NCHUNKS, CHUNK)
    x_p = jnp.pad(x, ((0, NPAD - N), (0, 0)))

    ones16 = jnp.ones((CHUNK, 16), f32)
    zeros16 = jnp.zeros((NPAD, 16), f32)
    zeros64 = jnp.zeros((NPAD, 64), f32)
    zeros128 = jnp.zeros((NPAD, 128), f32)

    d = _sc_deg(dst_p, ones16, zeros16)
    h0 = _tc_call(_tc_h0_body, jax.ShapeDtypeStruct((NPAD, 64), f32),
                  x_p, W_in, b_in.reshape(1, -1))
    dis, t1 = _tc_call(
        _tc_prep1_body,
        (jax.ShapeDtypeStruct((NPAD, 1), f32),
         jax.ShapeDtypeStruct((NPAD, 64), f32)),
        h0, d)

    apk = _sc_agg64(t1, src_p, dst_p, zeros64)
    t2 = _tc_call(_tc_mid1_body, jax.ShapeDtypeStruct((NPAD, 128), f32),
                  apk, t1, dis, W1, b1.reshape(1, -1), g1.reshape(1, -1),
                  be1.reshape(1, -1), W2)

    a0, a1 = _sc_agg128(t2, src_p, dst_p, zeros128)
    t3 = _tc_call(_tc_mid2_body, jax.ShapeDtypeStruct((NPAD, 64), f32),
                  a0, a1, t2, dis, b2.reshape(1, -1), g2.reshape(1, -1),
                  be2.reshape(1, -1), W3)

    apk = _sc_agg64(t3, src_p, dst_p, zeros64)
    out = _tc_call(_tc_final_body, jax.ShapeDtypeStruct((NPAD, 40), f32),
                   apk, t3, dis, b3.reshape(1, -1), g3.reshape(1, -1),
                   be3.reshape(1, -1), W_out, b_out.reshape(1, -1))
    return out[:N]


# speedup vs baseline: 1.0122x; 1.0122x over previous
import functools

import jax
import jax.numpy as jnp
from jax import lax
from jax.experimental import pallas as pl
from jax.experimental.pallas import tpu as pltpu
from jax.experimental.pallas import tpu_sc as plsc

N = 10000
NPAD = 10240
E = 320000
CHUNK = 128
NCHUNKS = 2560
CPW = NCHUNKS // 32
ROWS_PER_SUB = NPAD // 16

_MESH = plsc.VectorSubcoreMesh(core_axis_name="c", subcore_axis_name="s")
_SC_PARAMS = pltpu.CompilerParams(use_tc_tiling_on_sc=False)


RCHUNKS = E // CHUNK
DEG_CPW = 78
DEG_LAST = RCHUNKS - 31 * DEG_CPW


@functools.partial(
    pl.kernel,
    out_type=jax.ShapeDtypeStruct((NPAD, 128), jnp.float32),
    mesh=_MESH,
    compiler_params=_SC_PARAMS,
    scratch_types=[
        pltpu.VMEM_SHARED((NPAD, 16), jnp.float32),
        pltpu.VMEM((DEG_LAST, CHUNK), jnp.int32),
        pltpu.VMEM((CHUNK, 16), jnp.float32),
    ],
)
def _sc_deg(dst_hbm, ones_hbm, zeros_hbm, out, acc, idx_v, ones_v):
    cid = lax.axis_index("c")
    sid = lax.axis_index("s")
    sl = pl.ds(sid * ROWS_PER_SUB, ROWS_PER_SUB)
    pltpu.sync_copy(zeros_hbm.at[sl], acc.at[sl])
    pltpu.sync_copy(ones_hbm, ones_v)
    w = cid * 16 + sid
    plsc.subcore_barrier()

    @pl.when(w < 31)
    def _():
        pltpu.sync_copy(dst_hbm.at[pl.ds(w * DEG_CPW, DEG_CPW)],
                        idx_v.at[pl.ds(0, DEG_CPW)])

        @pl.loop(0, DEG_CPW)
        def _(i):
            pltpu.sync_copy(ones_v, acc.at[idx_v.at[i]], add=True)

    @pl.when(w == 31)
    def _():
        pltpu.sync_copy(dst_hbm.at[pl.ds(31 * DEG_CPW, DEG_LAST)], idx_v)

        @pl.loop(0, DEG_LAST)
        def _(i):
            pltpu.sync_copy(ones_v, acc.at[idx_v.at[i]], add=True)

    plsc.subcore_barrier()

    @pl.when(cid == 0)
    def _():
        pltpu.sync_copy(acc.at[sl], out.at[sl, pl.ds(0, 16)])

    @pl.when(cid == 1)
    def _():
        pltpu.sync_copy(acc.at[sl], out.at[sl, pl.ds(16, 16)])


BLK = 40
CPW0 = 80
CPW1 = 80


def _make_sc_agg(F, packed):
    out_type = (jax.ShapeDtypeStruct((NPAD, 2 * F), jnp.float32) if packed
                else (jax.ShapeDtypeStruct((NPAD, F), jnp.float32),
                      jax.ShapeDtypeStruct((NPAD, F), jnp.float32)))

    @functools.partial(
        pl.kernel,
        out_type=out_type,
        mesh=_MESH,
        compiler_params=_SC_PARAMS,
        scratch_types=[
            pltpu.VMEM_SHARED((NPAD, F), jnp.float32),
            pltpu.VMEM((BLK, CHUNK), jnp.int32),
            pltpu.VMEM((BLK, CHUNK), jnp.int32),
            pltpu.VMEM((CHUNK, F), jnp.float32),
            pltpu.VMEM((CHUNK, F), jnp.float32),
            pltpu.SemaphoreType.DMA,
            pltpu.SemaphoreType.DMA,
        ],
    )
    def k(tab_hbm, src_hbm, dst_hbm, zeros_hbm, *out_and_scratch):
        if packed:
            (out, acc, si_v, di_v, rows_a, rows_b,
             sem_a, sem_b) = out_and_scratch
        else:
            (out0, out1, acc, si_v, di_v, rows_a, rows_b,
             sem_a, sem_b) = out_and_scratch
        cid = lax.axis_index("c")
        sid = lax.axis_index("s")
        sl = pl.ds(sid * ROWS_PER_SUB, ROWS_PER_SUB)
        pltpu.sync_copy(zeros_hbm.at[sl], acc.at[sl])
        plsc.subcore_barrier()

        def run(worker_base, nblk):
            @pl.loop(0, nblk)
            def _(h):
                base = worker_base + h * BLK
                pltpu.sync_copy(src_hbm.at[pl.ds(base, BLK)], si_v)
                pltpu.sync_copy(dst_hbm.at[pl.ds(base, BLK)], di_v)
                pltpu.async_copy(tab_hbm.at[si_v.at[0]], rows_a, sem_a)

                @pl.loop(0, BLK, step=2)
                def _(i):
                    pltpu.async_copy(tab_hbm.at[si_v.at[i + 1]], rows_b, sem_b)
                    pltpu.make_async_copy(tab_hbm.at[si_v.at[i]], rows_a,
                                          sem_a).wait()
                    pltpu.sync_copy(rows_a, acc.at[di_v.at[i]], add=True)

                    @pl.when(i + 2 < BLK)
                    def _():
                        pltpu.async_copy(tab_hbm.at[si_v.at[i + 2]], rows_a,
                                         sem_a)

                    pltpu.make_async_copy(tab_hbm.at[si_v.at[i + 1]], rows_b,
                                          sem_b).wait()
                    pltpu.sync_copy(rows_b, acc.at[di_v.at[i + 1]], add=True)

        @pl.when(cid == 0)
        def _():
            run(sid * CPW0, CPW0 // BLK)

        @pl.when(cid == 1)
        def _():
            run(16 * CPW0 + sid * CPW1, CPW1 // BLK)

        plsc.subcore_barrier()

        if packed:
            @pl.when(cid == 0)
            def _():
                pltpu.sync_copy(acc.at[sl], out.at[sl, pl.ds(0, F)])

            @pl.when(cid == 1)
            def _():
                pltpu.sync_copy(acc.at[sl], out.at[sl, pl.ds(F, F)])
        else:
            @pl.when(cid == 0)
            def _():
                pltpu.sync_copy(acc.at[sl], out0.at[sl])

            @pl.when(cid == 1)
            def _():
                pltpu.sync_copy(acc.at[sl], out1.at[sl])

    return k


_sc_agg64 = _make_sc_agg(64, packed=True)
_sc_agg128 = _make_sc_agg(128, packed=False)


def _row_mask():
    return lax.broadcasted_iota(jnp.int32, (NPAD, 1), 0) < N


def _bn(z, g, b, mask, eps=1e-5):
    zm = jnp.where(mask, z, 0.0)
    m = jnp.sum(zm, axis=0, keepdims=True) * (1.0 / N)
    d = jnp.where(mask, z - m, 0.0)
    v = jnp.sum(d * d, axis=0, keepdims=True) * (1.0 / N)
    return (z - m) * lax.rsqrt(v + eps) * g + b


def _dot(a, b):
    return jnp.dot(a, b, preferred_element_type=jnp.float32)


def _tc_h0_body(x_ref, w_ref, b_ref, o_ref):
    o_ref[...] = jnp.maximum(_dot(x_ref[...], w_ref[...]) + b_ref[...], 0.0)


def _tc_prep1_body(h0_ref, d_ref, dis_ref, t1_ref):
    deg = d_ref[:, 0:1] + d_ref[:, 16:17] + 1.0
    dis = jnp.where(_row_mask(), lax.rsqrt(deg), 0.0)
    dis_ref[...] = dis
    t1_ref[...] = dis * h0_ref[...]


def _tc_mid1_body(apk, t1, dis_r, w1, b1, g1, be1, w2, t2_ref):
    mask = _row_mask()
    dis = dis_r[...]
    u = dis * (apk[:, :64] + apk[:, 64:] + t1[...])
    z = _dot(u, w1[...]) + b1[...]
    h = jnp.maximum(_bn(z, g1[...], be1[...], mask), 0.0)
    t2_ref[...] = dis * _dot(h, w2[...])


def _tc_mid2_body(a0, a1, t2, dis_r, b2, g2, be2, w3, t3_ref):
    mask = _row_mask()
    dis = dis_r[...]
    z = dis * (a0[...] + a1[...] + t2[...]) + b2[...]
    h = jnp.maximum(_bn(z, g2[...], be2[...], mask), 0.0)
    t3_ref[...] = dis * _dot(h, w3[...])


def _tc_final_body(apk, t3, dis_r, b3, g3, be3, wo, bo, out_ref):
    mask = _row_mask()
    dis = dis_r[...]
    z = dis * (apk[:, :64] + apk[:, 64:] + t3[...]) + b3[...]
    h = jnp.maximum(_bn(z, g3[...], be3[...], mask), 0.0)
    logits = _dot(h, wo[...]) + bo[...]
    m = jnp.max(logits, axis=1, keepdims=True)
    lse = m + jnp.log(jnp.sum(jnp.exp(logits - m), axis=1, keepdims=True))
    out_ref[...] = (logits - lse)[:N]


def _tc_call(body, out_shapes, *args):
    return pl.pallas_call(
        body,
        out_shape=out_shapes,
    )(*args)


def kernel(x, edge_index, W_in, b_in, W1, b1, g1, be1, W2, b2, g2, be2,
           W3, b3, g3, be3, W_out, b_out):
    f32 = jnp.float32
    pad = NCHUNKS * CHUNK - E
    fill = N + (jnp.arange(pad, dtype=jnp.int32) % CHUNK)
    src_p = jnp.concatenate([edge_index[0], fill]).reshape(NCHUNKS, CHUNK)
    dst_p = jnp.concatenate([edge_index[1], fill]).reshape(NCHUNKS, CHUNK)
    x_p = jnp.pad(x, ((0, NPAD - N), (0, 0)))

    ones16 = jnp.ones((CHUNK, 16), f32)
    zeros16 = jnp.zeros((NPAD, 16), f32)
    zeros64 = jnp.zeros((NPAD, 64), f32)
    zeros128 = jnp.zeros((NPAD, 128), f32)

    dst_raw = edge_index[1].reshape(RCHUNKS, CHUNK)
    d = _sc_deg(dst_raw, ones16, zeros16)
    h0 = _tc_call(_tc_h0_body, jax.ShapeDtypeStruct((NPAD, 64), f32),
                  x_p, W_in, b_in.reshape(1, -1))
    dis, t1 = pl.pallas_call(
        _tc_prep1_body,
        out_shape=(jax.ShapeDtypeStruct((NPAD, 1), f32),
                   jax.ShapeDtypeStruct((NPAD, 64), f32)),
        in_specs=[pl.BlockSpec((NPAD, 64), lambda: (0, 0)),
                  pl.BlockSpec((NPAD, 128), lambda: (0, 0))],
    )(h0, d)

    apk = _sc_agg64(t1, src_p, dst_p, zeros64)
    t2 = _tc_call(_tc_mid1_body, jax.ShapeDtypeStruct((NPAD, 128), f32),
                  apk, t1, dis, W1, b1.reshape(1, -1), g1.reshape(1, -1),
                  be1.reshape(1, -1), W2)

    a0, a1 = _sc_agg128(t2, src_p, dst_p, zeros128)
    t3 = _tc_call(_tc_mid2_body, jax.ShapeDtypeStruct((NPAD, 64), f32),
                  a0, a1, t2, dis, b2.reshape(1, -1), g2.reshape(1, -1),
                  be2.reshape(1, -1), W3)

    apk = _sc_agg64(t3, src_p, dst_p, zeros64)
    out = _tc_call(_tc_final_body, jax.ShapeDtypeStruct((N, 40), f32),
                   apk, t3, dis, b3.reshape(1, -1), g3.reshape(1, -1),
                   be3.reshape(1, -1), W_out, b_out.reshape(1, -1))
    return out
